# baseline (speedup 1.0000x reference)
import functools

import jax
import jax.numpy as jnp
from jax import lax
from jax.experimental import pallas as pl
from jax.experimental.pallas import tpu as pltpu
from jax.experimental.pallas import tpu_sc as plsc

N_NODES = 10000
N_EDGES = 320000
D = 128

NC = 2
NS = 16
NW = NC * NS

CHUNK = 40
NCHUNK = 250
EDGES_PER_W = NCHUNK * CHUNK
NBUF = 5
GD = 4

N_PAD = 10240
ROWS_PER_TILE = N_PAD // NS


def _matmul_norm_kernel(h_ref, w_ref, norm_ref, out_ref, z_ref):
    out_ref[...] = (
        jnp.dot(h_ref[...], w_ref[...], preferred_element_type=jnp.float32)
        * norm_ref[...]
    )
    z_ref[...] = jnp.zeros_like(z_ref)


def _matmul_norm(h, W, norm):
    nb = 10
    bs = N_NODES // nb
    zbs = N_PAD // nb
    return pl.pallas_call(
        _matmul_norm_kernel,
        grid=(nb,),
        in_specs=[
            pl.BlockSpec((bs, D), lambda i: (i, 0)),
            pl.BlockSpec((D, D), lambda i: (0, 0)),
            pl.BlockSpec((bs, 1), lambda i: (i, 0)),
        ],
        out_specs=[
            pl.BlockSpec((bs, D), lambda i: (i, 0)),
            pl.BlockSpec((zbs, D), lambda i: (i, 0)),
        ],
        out_shape=[
            jax.ShapeDtypeStruct((N_NODES, D), jnp.float32),
            jax.ShapeDtypeStruct((N_PAD, D), jnp.float32),
        ],
    )(h, W, norm)


def _finish_kernel(p_ref, b_ref, out_ref):
    out_ref[...] = jnp.maximum(p_ref[0] + p_ref[1] + b_ref[...], 0.0)


def _finish(partials, b):
    nb = 10
    bs = N_NODES // nb
    return pl.pallas_call(
        _finish_kernel,
        grid=(nb,),
        in_specs=[
            pl.BlockSpec((NC, bs, D), lambda i: (0, i, 0)),
            pl.BlockSpec((1, D), lambda i: (0, 0)),
        ],
        out_specs=pl.BlockSpec((bs, D), lambda i: (i, 0)),
        out_shape=jax.ShapeDtypeStruct((N_NODES, D), jnp.float32),
    )(partials, b)


def _sc_scatter(normh, src_r, dst_r, zeros):
    mesh = plsc.VectorSubcoreMesh(core_axis_name="c", subcore_axis_name="s")

    @functools.partial(
        pl.kernel,
        mesh=mesh,
        out_type=jax.ShapeDtypeStruct((NC, N_PAD, D), jnp.float32),
        scratch_types=[
            pltpu.VMEM((NCHUNK * CHUNK,), jnp.int32),
            pltpu.VMEM((NCHUNK * CHUNK,), jnp.int32),
            [pltpu.VMEM((CHUNK, D), jnp.float32) for _ in range(NBUF)],
            pltpu.VMEM_SHARED((N_PAD, D), jnp.float32),
            [pltpu.SemaphoreType.DMA for _ in range(NBUF)],
            [pltpu.SemaphoreType.DMA for _ in range(NBUF)],
        ],
    )
    def k(normh_hbm, src_hbm, dst_hbm, zeros_hbm, out_hbm,
          src_v, dst_v, rows, acc, gsem, ssem):
        cid = lax.axis_index("c")
        sid = lax.axis_index("s")
        wid = cid * NS + sid

        pltpu.sync_copy(src_hbm.at[wid], src_v)
        pltpu.sync_copy(dst_hbm.at[wid], dst_v)
        pltpu.sync_copy(
            zeros_hbm.at[pl.ds(sid * ROWS_PER_TILE, ROWS_PER_TILE)],
            acc.at[pl.ds(sid * ROWS_PER_TILE, ROWS_PER_TILE)],
        )
        plsc.subcore_barrier()

        HALVES = ((0, 24), (24, 16))

        def gather_start(j, b):
            for o, n in HALVES:
                pltpu.async_copy(
                    normh_hbm.at[src_v.at[pl.ds(j * CHUNK + o, n)]],
                    rows[b].at[pl.ds(o, n)], gsem[b])

        def gather_wait(j, b):
            for o, n in HALVES:
                pltpu.make_async_copy(
                    normh_hbm.at[src_v.at[pl.ds(j * CHUNK + o, n)]],
                    rows[b].at[pl.ds(o, n)], gsem[b]).wait()

        def scatter_start(j, b):
            pltpu.async_copy(rows[b], acc.at[dst_v.at[pl.ds(j * CHUNK, CHUNK)]],
                             ssem[b], add=True)

        def scatter_wait(j, b):
            pltpu.make_async_copy(rows[b],
                                  acc.at[dst_v.at[pl.ds(j * CHUNK, CHUNK)]],
                                  ssem[b]).wait()

        for j in range(GD):
            gather_start(j, j)
        for j in range(NBUF):
            gather_wait(j, j)
            scatter_start(j, j)
            if j >= NBUF - GD:
                scatter_wait(j - (NBUF - GD), (j - (NBUF - GD)) % NBUF)
            gather_start(j + GD, (j + GD) % NBUF)

        def steady(i, carry):
            for u in range(NBUF):
                j = i * NBUF + u
                gather_wait(j, u)
                scatter_start(j, u)
                scatter_wait(j - (NBUF - GD), (u - (NBUF - GD)) % NBUF)
                gather_start(j + GD, (u + GD) % NBUF)
            return carry

        lax.fori_loop(1, NCHUNK // NBUF - 1, steady, 0)

        for j in range(NCHUNK - NBUF, NCHUNK):
            u = j % NBUF
            gather_wait(j, u)
            scatter_start(j, u)
            scatter_wait(j - (NBUF - GD), (u - (NBUF - GD)) % NBUF)
            if j + GD < NCHUNK:
                gather_start(j + GD, (u + GD) % NBUF)
        for j in range(NCHUNK - (NBUF - GD), NCHUNK):
            scatter_wait(j, j % NBUF)

        plsc.subcore_barrier()
        pltpu.sync_copy(
            acc.at[pl.ds(sid * ROWS_PER_TILE, ROWS_PER_TILE)],
            out_hbm.at[cid, pl.ds(sid * ROWS_PER_TILE, ROWS_PER_TILE)],
        )

    return k(normh, src_r, dst_r, zeros)


def kernel(h, edge_index, norm, W, b):
    normh, zeros = _matmul_norm(h, W, norm)
    src_r = edge_index[0].reshape(NW, EDGES_PER_W)
    dst_r = edge_index[1].reshape(NW, EDGES_PER_W)
    partials = _sc_scatter(normh, src_r, dst_r, zeros)
    return _finish(partials, b.reshape(1, D))


# speedup vs baseline: 1.0035x; 1.0035x over previous
import functools

import jax
import jax.numpy as jnp
from jax import lax
from jax.experimental import pallas as pl
from jax.experimental.pallas import tpu as pltpu
from jax.experimental.pallas import tpu_sc as plsc

N_NODES = 10000
N_EDGES = 320000
D = 128

NC = 2
NS = 16
NW = NC * NS

CHUNK = 40
NCHUNK = 250
EDGES_PER_W = NCHUNK * CHUNK
NBUF = 5
GD = 4

N_PAD = 10240
ROWS_PER_TILE = N_PAD // NS


def _matmul_norm_kernel(h_ref, w_ref, norm_ref, out_ref, z_ref):
    out_ref[...] = (
        jnp.dot(h_ref[...], w_ref[...], preferred_element_type=jnp.float32)
        * norm_ref[...]
    )
    z_ref[...] = jnp.zeros_like(z_ref)


def _matmul_norm(h, W, norm):
    nb = 10
    bs = N_NODES // nb
    zbs = N_PAD // nb
    return pl.pallas_call(
        _matmul_norm_kernel,
        grid=(nb,),
        in_specs=[
            pl.BlockSpec((bs, D), lambda i: (i, 0)),
            pl.BlockSpec((D, D), lambda i: (0, 0)),
            pl.BlockSpec((bs, 1), lambda i: (i, 0)),
        ],
        out_specs=[
            pl.BlockSpec((bs, D), lambda i: (i, 0)),
            pl.BlockSpec((zbs, D), lambda i: (i, 0)),
        ],
        out_shape=[
            jax.ShapeDtypeStruct((N_NODES, D), jnp.float32),
            jax.ShapeDtypeStruct((N_PAD, D), jnp.float32),
        ],
    )(h, W, norm)


def _finish_kernel(p_ref, b_ref, out_ref):
    out_ref[...] = jnp.maximum(p_ref[0] + p_ref[1] + b_ref[...], 0.0)


def _finish(partials, b):
    nb = 10
    bs = N_NODES // nb
    return pl.pallas_call(
        _finish_kernel,
        grid=(nb,),
        in_specs=[
            pl.BlockSpec((NC, bs, D), lambda i: (0, i, 0)),
            pl.BlockSpec((1, D), lambda i: (0, 0)),
        ],
        out_specs=pl.BlockSpec((bs, D), lambda i: (i, 0)),
        out_shape=jax.ShapeDtypeStruct((N_NODES, D), jnp.float32),
    )(partials, b)


def _sc_scatter(normh, src_r, dst_r, zeros):
    mesh = plsc.VectorSubcoreMesh(core_axis_name="c", subcore_axis_name="s")

    @functools.partial(
        pl.kernel,
        mesh=mesh,
        out_type=jax.ShapeDtypeStruct((NC, N_PAD, D), jnp.float32),
        scratch_types=[
            pltpu.VMEM((NCHUNK * CHUNK,), jnp.int32),
            pltpu.VMEM((NCHUNK * CHUNK,), jnp.int32),
            [pltpu.VMEM((CHUNK, D), jnp.float32) for _ in range(NBUF)],
            pltpu.VMEM_SHARED((N_PAD, D), jnp.float32),
            [pltpu.SemaphoreType.DMA for _ in range(NBUF)],
            [pltpu.SemaphoreType.DMA for _ in range(NBUF)],
        ],
    )
    def k(normh_hbm, src_hbm, dst_hbm, zeros_hbm, out_hbm,
          src_v, dst_v, rows, acc, gsem, ssem):
        cid = lax.axis_index("c")
        sid = lax.axis_index("s")
        wid = cid * NS + sid

        pltpu.sync_copy(src_hbm.at[wid], src_v)
        pltpu.sync_copy(dst_hbm.at[wid], dst_v)
        pltpu.sync_copy(
            zeros_hbm.at[pl.ds(sid * ROWS_PER_TILE, ROWS_PER_TILE)],
            acc.at[pl.ds(sid * ROWS_PER_TILE, ROWS_PER_TILE)],
        )
        plsc.subcore_barrier()

        def gather_start(j, b):
            pltpu.async_copy(normh_hbm.at[src_v.at[pl.ds(j * CHUNK, CHUNK)]],
                             rows[b], gsem[b])

        def gather_wait(j, b):
            pltpu.make_async_copy(
                normh_hbm.at[src_v.at[pl.ds(j * CHUNK, CHUNK)]], rows[b],
                gsem[b]).wait()

        def scatter_start(j, b):
            pltpu.async_copy(rows[b], acc.at[dst_v.at[pl.ds(j * CHUNK, CHUNK)]],
                             ssem[b], add=True)

        def scatter_wait(j, b):
            pltpu.make_async_copy(rows[b],
                                  acc.at[dst_v.at[pl.ds(j * CHUNK, CHUNK)]],
                                  ssem[b]).wait()

        for j in range(GD):
            gather_start(j, j)
        for j in range(NBUF):
            gather_wait(j, j)
            scatter_start(j, j)
            if j >= NBUF - GD:
                scatter_wait(j - (NBUF - GD), (j - (NBUF - GD)) % NBUF)
            gather_start(j + GD, (j + GD) % NBUF)

        def steady(i, carry):
            for u in range(NBUF):
                j = i * NBUF + u
                gather_wait(j, u)
                scatter_start(j, u)
                scatter_wait(j - (NBUF - GD), (u - (NBUF - GD)) % NBUF)
                gather_start(j + GD, (u + GD) % NBUF)
            return carry

        lax.fori_loop(1, NCHUNK // NBUF - 1, steady, 0)

        for j in range(NCHUNK - NBUF, NCHUNK):
            u = j % NBUF
            gather_wait(j, u)
            scatter_start(j, u)
            scatter_wait(j - (NBUF - GD), (u - (NBUF - GD)) % NBUF)
            if j + GD < NCHUNK:
                gather_start(j + GD, (u + GD) % NBUF)
        for j in range(NCHUNK - (NBUF - GD), NCHUNK):
            scatter_wait(j, j % NBUF)

        plsc.subcore_barrier()
        pltpu.sync_copy(
            acc.at[pl.ds(sid * ROWS_PER_TILE, ROWS_PER_TILE)],
            out_hbm.at[cid, pl.ds(sid * ROWS_PER_TILE, ROWS_PER_TILE)],
        )

    return k(normh, src_r, dst_r, zeros)


def kernel(h, edge_index, norm, W, b):
    normh, zeros = _matmul_norm(h, W, norm)
    src_r = edge_index[0].reshape(NW, EDGES_PER_W)
    dst_r = edge_index[1].reshape(NW, EDGES_PER_W)
    partials = _sc_scatter(normh, src_r, dst_r, zeros)
    return _finish(partials, b.reshape(1, D))
